# baseline (speedup 1.0000x reference)
import functools

import jax
import jax.numpy as jnp
from jax import lax
from jax.experimental import pallas as pl
from jax.experimental.pallas import tpu as pltpu
from jax.experimental.pallas import tpu_sc as plsc

ND, NG, DIN, HID, OUTD = 10000, 50000, 128, 64, 32
NDP, NGP = 10240, 50176
EDG = 160000
NCORE, NSUB = 2, 16
IW = 128
EPAD = 163840
IROWS = EPAD // IW
RPT = IROWS // NSUB
CHR = 5
NCH = RPT // CHR


def _enc_body(x_ref, w_ref, b_ref, *outs):
    y = jnp.dot(x_ref[...], w_ref[...], preferred_element_type=jnp.float32)
    y = y + b_ref[...]
    for g in range(8):
        outs[g][...] = y[:, 32 * g:32 * (g + 1)]


def _encode_tables(x, wcat, bcat, n, blk):
    nb = n // blk
    return pl.pallas_call(
        _enc_body,
        grid=(nb,),
        in_specs=[
            pl.BlockSpec((blk, DIN), lambda i: (i, 0)),
            pl.BlockSpec((DIN, 256), lambda i: (0, 0)),
            pl.BlockSpec((1, 256), lambda i: (0, 0)),
        ],
        out_specs=[pl.BlockSpec((blk, 32), lambda i: (i, 0))] * 8,
        out_shape=[jax.ShapeDtypeStruct((n, 32), jnp.float32)] * 8,
    )(x, wcat, bcat)


@functools.partial(
    pl.kernel,
    out_type=[jax.ShapeDtypeStruct((NDP, 32), jnp.float32)] * 4
             + [jax.ShapeDtypeStruct((NGP, 32), jnp.float32)] * 4,
    mesh=plsc.VectorSubcoreMesh(
        core_axis_name="c", subcore_axis_name="s",
        num_cores=NCORE, num_subcores=NSUB),
    scratch_types=[
        pltpu.VMEM((CHR, IW), jnp.int32),
        pltpu.VMEM((CHR, IW), jnp.int32),
        pltpu.VMEM((CHR * IW, 32), jnp.float32),
        pltpu.VMEM_SHARED((NGP, 32), jnp.float32),
        pltpu.SemaphoreType.DMA,
        pltpu.SemaphoreType.DMA,
    ],
    compiler_params=pltpu.CompilerParams(use_tc_tiling_on_sc=False, internal_scratch_in_bytes=131072),
)
def _sc_segsum(zeros_hbm,
               t_dd_lo, t_dd_hi, t_gd_lo, t_gd_hi,
               t_dg_lo, t_dg_hi, t_gg_lo, t_gg_hi,
               s_dd, d_dd, s_gd, d_gd, s_dg, d_dg, s_gg, d_gg,
               a_dd_lo, a_dd_hi, a_gd_lo, a_gd_hi,
               a_dg_lo, a_dg_hi, a_gg_lo, a_gg_hi,
               sidx, didx, rows, acc, semg, sems):
    c = lax.axis_index("c")
    t = lax.axis_index("s")

    ets = [
        ((t_dd_lo, t_dd_hi), s_dd, d_dd, (a_dd_lo, a_dd_hi), NDP),
        ((t_gd_lo, t_gd_hi), s_gd, d_gd, (a_gd_lo, a_gd_hi), NDP),
        ((t_dg_lo, t_dg_hi), s_dg, d_dg, (a_dg_lo, a_dg_hi), NGP),
        ((t_gg_lo, t_gg_hi), s_gg, d_gg, (a_gg_lo, a_gg_hi), NGP),
    ]
    for tabs, s_hbm, d_hbm, aggs, ndst in ets:
        nrt = ndst // NSUB
        lo = t * nrt
        pltpu.sync_copy(zeros_hbm.at[pl.ds(lo, nrt)], acc.at[pl.ds(lo, nrt)])
        plsc.subcore_barrier()
        for half in range(NCORE):
            @pl.when(c == half)
            def _(tab=tabs[half]):
                def chunk(ci, carry):
                    row0 = t * RPT + ci * CHR
                    pltpu.sync_copy(s_hbm.at[pl.ds(row0, CHR)], sidx)
                    pltpu.sync_copy(d_hbm.at[pl.ds(row0, CHR)], didx)
                    cps = [pltpu.async_copy(tab.at[sidx.at[r]],
                                            rows.at[pl.ds(r * IW, IW)], semg)
                           for r in range(CHR)]
                    for cp in cps:
                        cp.wait()
                    cps = [pltpu.async_copy(rows.at[pl.ds(r * IW, IW)],
                                            acc.at[didx.at[r]], sems, add=True)
                           for r in range(CHR)]
                    for cp in cps:
                        cp.wait()
                    return carry
                lax.fori_loop(0, NCH, chunk, 0)
        plsc.subcore_barrier()
        for half in range(NCORE):
            @pl.when(c == half)
            def _(agg=aggs[half]):
                pltpu.sync_copy(acc.at[pl.ds(lo, nrt)], agg.at[pl.ds(lo, nrt)])
        plsc.subcore_barrier()


def _dec_body(aAlo_r, aAhi_r, aBlo_r, aBhi_r, sAlo_r, sAhi_r, sBlo_r, sBhi_r,
              mw1_r, mb1_r, mw2_r, mb2_r, lw1_r, lb1_r, lw2_r, lb2_r,
              mu_ref, ls_ref):
    u = jnp.concatenate([aAlo_r[...] + sAlo_r[...],
                         aAhi_r[...] + sAhi_r[...]], axis=1)
    v = jnp.concatenate([aBlo_r[...] + sBlo_r[...],
                         aBhi_r[...] + sBhi_r[...]], axis=1)
    ones = jnp.ones((HID, HID), jnp.float32)
    n2u = jnp.dot(u * u, ones, preferred_element_type=jnp.float32)
    n2v = jnp.dot(v * v, ones, preferred_element_type=jnp.float32)
    h = (u * lax.rsqrt(jnp.maximum(n2u, 1e-24))
         + v * lax.rsqrt(jnp.maximum(n2v, 1e-24)))
    hm = jnp.tanh(jnp.dot(h, mw1_r[...], preferred_element_type=jnp.float32)
                  + mb1_r[...])
    mu_ref[...] = jnp.dot(hm, mw2_r[...],
                          preferred_element_type=jnp.float32) + mb2_r[...]
    hl = jnp.tanh(jnp.dot(h, lw1_r[...], preferred_element_type=jnp.float32)
                  + lb1_r[...])
    ls_ref[...] = jnp.minimum(
        jnp.dot(hl, lw2_r[...], preferred_element_type=jnp.float32)
        + lb2_r[...], 10.0)


def _decode(aAlo, aAhi, aBlo, aBhi, sAlo, sAhi, sBlo, sBhi,
            mw1, mb1, mw2, mb2, lw1, lb1, lw2, lb2, n, blk):
    nb = n // blk
    big = pl.BlockSpec((blk, 32), lambda i: (i, 0))
    full = lambda shape: pl.BlockSpec(shape, lambda i: (0, 0))
    return pl.pallas_call(
        _dec_body,
        grid=(nb,),
        in_specs=[big] * 8 + [
            full((HID, 16)), full((1, 16)), full((16, OUTD)), full((1, OUTD)),
            full((HID, 16)), full((1, 16)), full((16, OUTD)), full((1, OUTD)),
        ],
        out_specs=[big, big],
        out_shape=[jax.ShapeDtypeStruct((n, OUTD), jnp.float32)] * 2,
    )(aAlo, aAhi, aBlo, aBhi, sAlo, sAhi, sBlo, sBhi,
      mw1, mb1.reshape(1, 16), mw2, mb2.reshape(1, OUTD),
      lw1, lb1.reshape(1, 16), lw2, lb2.reshape(1, OUTD))


def kernel(x_drug, x_gene, ei_dd, ei_dg, ei_gd, ei_gg,
           Wm_dd, bm_dd, Ws_dd, bs_dd, Wm_dg, bm_dg, Ws_dg, bs_dg,
           Wm_gd, bm_gd, Ws_gd, bs_gd, Wm_gg, bm_gg, Ws_gg, bs_gg,
           mu_W1_drug, mu_b1_drug, mu_W2_drug, mu_b2_drug,
           ls_W1_drug, ls_b1_drug, ls_W2_drug, ls_b2_drug,
           mu_W1_gene, mu_b1_gene, mu_W2_gene, mu_b2_gene,
           ls_W1_gene, ls_b1_gene, ls_W2_gene, ls_b2_gene):
    wcat_d = jnp.concatenate([Wm_dd, Wm_dg, Ws_dd, Ws_gd], axis=1)
    bcat_d = jnp.concatenate([bm_dd, bm_dg, bs_dd, bs_gd]).reshape(1, 256)
    wcat_g = jnp.concatenate([Wm_gd, Wm_gg, Ws_dg, Ws_gg], axis=1)
    bcat_g = jnp.concatenate([bm_gd, bm_gg, bs_dg, bs_gg]).reshape(1, 256)
    td = _encode_tables(x_drug, wcat_d, bcat_d, ND, 1000)
    tg = _encode_tables(x_gene, wcat_g, bcat_g, NG, 1000)

    zeros = jnp.zeros((NGP, 32), jnp.float32)
    ar = jnp.arange(EPAD - EDG, dtype=jnp.int32)

    def pidx(ei, nsrc, ndst_real):
        s = jnp.concatenate([ei[0], ar % nsrc]).reshape(IROWS, IW)
        dd = jnp.concatenate([ei[1], ndst_real + ar % 64]).reshape(IROWS, IW)
        return s, dd

    s_dd, d_dd = pidx(ei_dd, ND, ND)
    s_gd, d_gd = pidx(ei_gd, NG, ND)
    s_dg, d_dg = pidx(ei_dg, ND, NG)
    s_gg, d_gg = pidx(ei_gg, NG, NG)
    (a_dd_lo, a_dd_hi, a_gd_lo, a_gd_hi,
     a_dg_lo, a_dg_hi, a_gg_lo, a_gg_hi) = _sc_segsum(
        zeros,
        td[0], td[1], tg[0], tg[1], td[2], td[3], tg[2], tg[3],
        s_dd, d_dd, s_gd, d_gd, s_dg, d_dg, s_gg, d_gg)

    mu_d, ls_d = _decode(a_dd_lo, a_dd_hi, a_gd_lo, a_gd_hi,
                         td[4], td[5], td[6], td[7],
                         mu_W1_drug, mu_b1_drug, mu_W2_drug, mu_b2_drug,
                         ls_W1_drug, ls_b1_drug, ls_W2_drug, ls_b2_drug,
                         ND, 5000)
    mu_g, ls_g = _decode(a_dg_lo, a_dg_hi, a_gg_lo, a_gg_hi,
                         tg[4], tg[5], tg[6], tg[7],
                         mu_W1_gene, mu_b1_gene, mu_W2_gene, mu_b2_gene,
                         ls_W1_gene, ls_b1_gene, ls_W2_gene, ls_b2_gene,
                         NG, 5000)
    return (mu_d, mu_g, mu_d, mu_g, ls_d, ls_g)


# speedup vs baseline: 1.2230x; 1.2230x over previous
import functools

import jax
import jax.numpy as jnp
from jax import lax
from jax.experimental import pallas as pl
from jax.experimental.pallas import tpu as pltpu
from jax.experimental.pallas import tpu_sc as plsc

ND, NG, DIN, HID, OUTD = 10000, 50000, 128, 64, 32
NDP, NGP = 10240, 50176
EDG = 160000
NCORE, NSUB = 2, 16
IW = 128
EPAD = 163840
IROWS = EPAD // IW
RPT = IROWS // NSUB
CHR = 5
NCH = RPT // CHR


def _enc_body(x_ref, w_ref, b_ref, m1lo, m1hi, m2lo, m2hi, skp):
    y = jnp.dot(x_ref[...], w_ref[...], preferred_element_type=jnp.float32)
    y = y + b_ref[...]
    m1lo[...] = y[:, 0:32]
    m1hi[...] = y[:, 32:64]
    m2lo[...] = y[:, 64:96]
    m2hi[...] = y[:, 96:128]
    skp[...] = y[:, 128:256]


def _encode_tables(x, wcat, bcat, n, blk):
    nb = n // blk
    return pl.pallas_call(
        _enc_body,
        grid=(nb,),
        in_specs=[
            pl.BlockSpec((blk, DIN), lambda i: (i, 0)),
            pl.BlockSpec((DIN, 256), lambda i: (0, 0)),
            pl.BlockSpec((1, 256), lambda i: (0, 0)),
        ],
        out_specs=[pl.BlockSpec((blk, 32), lambda i: (i, 0))] * 4
                  + [pl.BlockSpec((blk, 128), lambda i: (i, 0))],
        out_shape=[jax.ShapeDtypeStruct((n, 32), jnp.float32)] * 4
                  + [jax.ShapeDtypeStruct((n, 128), jnp.float32)],
    )(x, wcat, bcat)


@functools.partial(
    pl.kernel,
    out_type=[jax.ShapeDtypeStruct((NDP, 128), jnp.float32),
              jax.ShapeDtypeStruct((NGP, 128), jnp.float32)],
    mesh=plsc.VectorSubcoreMesh(
        core_axis_name="c", subcore_axis_name="s",
        num_cores=NCORE, num_subcores=NSUB),
    scratch_types=[
        pltpu.VMEM((CHR, IW), jnp.int32),
        pltpu.VMEM((CHR, IW), jnp.int32),
        pltpu.VMEM((CHR * IW, 32), jnp.float32),
        pltpu.VMEM_SHARED((NGP, 32), jnp.float32),
        pltpu.SemaphoreType.DMA,
        pltpu.SemaphoreType.DMA,
    ],
    compiler_params=pltpu.CompilerParams(use_tc_tiling_on_sc=False),
)
def _sc_segsum(zeros_hbm,
               t_dd_lo, t_dd_hi, t_gd_lo, t_gd_hi,
               t_dg_lo, t_dg_hi, t_gg_lo, t_gg_hi,
               s_dd, d_dd, s_gd, d_gd, s_dg, d_dg, s_gg, d_gg,
               ap_drug, ap_gene,
               sidx, didx, rows, acc, semg, sems):
    c = lax.axis_index("c")
    t = lax.axis_index("s")

    ets = [
        ((t_dd_lo, t_dd_hi), s_dd, d_dd, ap_drug, 0, NDP),
        ((t_gd_lo, t_gd_hi), s_gd, d_gd, ap_drug, 64, NDP),
        ((t_dg_lo, t_dg_hi), s_dg, d_dg, ap_gene, 0, NGP),
        ((t_gg_lo, t_gg_hi), s_gg, d_gg, ap_gene, 64, NGP),
    ]
    for tabs, s_hbm, d_hbm, ap, cbase, ndst in ets:
        nrt = ndst // NSUB
        lo = t * nrt
        pltpu.sync_copy(zeros_hbm.at[pl.ds(lo, nrt)], acc.at[pl.ds(lo, nrt)])
        plsc.subcore_barrier()
        for half in range(NCORE):
            @pl.when(c == half)
            def _(tab=tabs[half]):
                def chunk(ci, carry):
                    row0 = t * RPT + ci * CHR
                    pltpu.sync_copy(s_hbm.at[pl.ds(row0, CHR)], sidx)
                    pltpu.sync_copy(d_hbm.at[pl.ds(row0, CHR)], didx)
                    cps = [pltpu.async_copy(tab.at[sidx.at[r]],
                                            rows.at[pl.ds(r * IW, IW)], semg)
                           for r in range(CHR)]
                    for cp in cps:
                        cp.wait()
                    cps = [pltpu.async_copy(rows.at[pl.ds(r * IW, IW)],
                                            acc.at[didx.at[r]], sems, add=True)
                           for r in range(CHR)]
                    for cp in cps:
                        cp.wait()
                    return carry
                lax.fori_loop(0, NCH, chunk, 0)
        plsc.subcore_barrier()
        for half in range(NCORE):
            @pl.when(c == half)
            def _(ap=ap, col=cbase + 32 * half):
                pltpu.sync_copy(acc.at[pl.ds(lo, nrt)],
                                ap.at[pl.ds(lo, nrt), pl.ds(col, 32)])
        plsc.subcore_barrier()


def _dec_body(ap_r, sp_r, mw1_r, mb1_r, mw2_r, mb2_r,
              lw1_r, lb1_r, lw2_r, lb2_r, mu_ref, ls_ref):
    w = ap_r[...] + sp_r[...]
    ri = lax.broadcasted_iota(jnp.int32, (2 * HID, 2 * HID), 0) // HID
    ci = lax.broadcasted_iota(jnp.int32, (2 * HID, 2 * HID), 1) // HID
    bd = (ri == ci).astype(jnp.float32)
    n2 = jnp.dot(w * w, bd, preferred_element_type=jnp.float32)
    hh = w * lax.rsqrt(jnp.maximum(n2, 1e-24))
    hm = jnp.tanh(jnp.dot(hh, mw1_r[...], preferred_element_type=jnp.float32)
                  + mb1_r[...])
    mu_ref[...] = jnp.dot(hm, mw2_r[...],
                          preferred_element_type=jnp.float32) + mb2_r[...]
    hl = jnp.tanh(jnp.dot(hh, lw1_r[...], preferred_element_type=jnp.float32)
                  + lb1_r[...])
    ls_ref[...] = jnp.minimum(
        jnp.dot(hl, lw2_r[...], preferred_element_type=jnp.float32)
        + lb2_r[...], 10.0)


def _decode(ap, sp, mw1, mb1, mw2, mb2, lw1, lb1, lw2, lb2, n, blk):
    nb = n // blk
    big = lambda width: pl.BlockSpec((blk, width), lambda i: (i, 0))
    full = lambda shape: pl.BlockSpec(shape, lambda i: (0, 0))
    mw12 = jnp.concatenate([mw1, mw1], axis=0)
    lw12 = jnp.concatenate([lw1, lw1], axis=0)
    return pl.pallas_call(
        _dec_body,
        grid=(nb,),
        in_specs=[big(128), big(128),
                  full((2 * HID, 16)), full((1, 16)),
                  full((16, OUTD)), full((1, OUTD)),
                  full((2 * HID, 16)), full((1, 16)),
                  full((16, OUTD)), full((1, OUTD))],
        out_specs=[big(OUTD), big(OUTD)],
        out_shape=[jax.ShapeDtypeStruct((n, OUTD), jnp.float32)] * 2,
    )(ap, sp,
      mw12, mb1.reshape(1, 16), mw2, mb2.reshape(1, OUTD),
      lw12, lb1.reshape(1, 16), lw2, lb2.reshape(1, OUTD))


def kernel(x_drug, x_gene, ei_dd, ei_dg, ei_gd, ei_gg,
           Wm_dd, bm_dd, Ws_dd, bs_dd, Wm_dg, bm_dg, Ws_dg, bs_dg,
           Wm_gd, bm_gd, Ws_gd, bs_gd, Wm_gg, bm_gg, Ws_gg, bs_gg,
           mu_W1_drug, mu_b1_drug, mu_W2_drug, mu_b2_drug,
           ls_W1_drug, ls_b1_drug, ls_W2_drug, ls_b2_drug,
           mu_W1_gene, mu_b1_gene, mu_W2_gene, mu_b2_gene,
           ls_W1_gene, ls_b1_gene, ls_W2_gene, ls_b2_gene):
    wcat_d = jnp.concatenate([Wm_dd, Wm_dg, Ws_dd, Ws_gd], axis=1)
    bcat_d = jnp.concatenate([bm_dd, bm_dg, bs_dd, bs_gd]).reshape(1, 256)
    wcat_g = jnp.concatenate([Wm_gd, Wm_gg, Ws_dg, Ws_gg], axis=1)
    bcat_g = jnp.concatenate([bm_gd, bm_gg, bs_dg, bs_gg]).reshape(1, 256)
    td = _encode_tables(x_drug, wcat_d, bcat_d, ND, 1000)
    tg = _encode_tables(x_gene, wcat_g, bcat_g, NG, 1000)

    zeros = jnp.zeros((NGP, 32), jnp.float32)
    ar = jnp.arange(EPAD - EDG, dtype=jnp.int32)

    def pidx(ei, nsrc, ndst_real):
        s = jnp.concatenate([ei[0], ar % nsrc]).reshape(IROWS, IW)
        dd = jnp.concatenate([ei[1], ndst_real + ar % 64]).reshape(IROWS, IW)
        return s, dd

    s_dd, d_dd = pidx(ei_dd, ND, ND)
    s_gd, d_gd = pidx(ei_gd, NG, ND)
    s_dg, d_dg = pidx(ei_dg, ND, NG)
    s_gg, d_gg = pidx(ei_gg, NG, NG)
    ap_drug, ap_gene = _sc_segsum(
        zeros,
        td[0], td[1], tg[0], tg[1], td[2], td[3], tg[2], tg[3],
        s_dd, d_dd, s_gd, d_gd, s_dg, d_dg, s_gg, d_gg)

    mu_d, ls_d = _decode(ap_drug, td[4],
                         mu_W1_drug, mu_b1_drug, mu_W2_drug, mu_b2_drug,
                         ls_W1_drug, ls_b1_drug, ls_W2_drug, ls_b2_drug,
                         ND, 5000)
    mu_g, ls_g = _decode(ap_gene, tg[4],
                         mu_W1_gene, mu_b1_gene, mu_W2_gene, mu_b2_gene,
                         ls_W1_gene, ls_b1_gene, ls_W2_gene, ls_b2_gene,
                         NG, 5000)
    return (mu_d, mu_g, mu_d, mu_g, ls_d, ls_g)
